# baseline (speedup 1.0000x reference)
import jax
import jax.numpy as jnp
from jax.experimental import pallas as pl
from jax.experimental.pallas import tpu as pltpu

INPUT_DIM = 1024
LATENT_DIM = 4096
K = 64
TM = 256
NCHUNK = 64
WINDOW = 1 << 16

INT_MIN = -(2**31)


def _mxu_count(mask, ones_f):
    m = mask.astype(jnp.float32)
    cnt = jax.lax.dot_general(
        m, ones_f, (((1,), (0,)), ((), ())),
        preferred_element_type=jnp.float32,
    )[:, 0:1]
    return cnt.astype(jnp.int32)


def _body(x_ref, we_ref, be_ref, wd_ref, bd_ref, sp_ref, rec_ref):
    lat = jax.lax.dot_general(
        x_ref[...], we_ref[...], (((1,), (1,)), ((), ())),
        preferred_element_type=jnp.float32,
    ) + be_ref[...]

    ikey = jax.lax.bitcast_convert_type(lat, jnp.int32)
    key = jnp.where(ikey < 0, ikey ^ jnp.int32(0x7FFFFFFF), ikey)

    ones_bf = jnp.ones((LATENT_DIM, 128), dtype=jnp.float32)

    lo0 = jnp.full((TM, 1), INT_MIN, dtype=jnp.int32)
    hi0 = jnp.full((TM, 1), 2**31 - 1, dtype=jnp.int32)

    def bis_step(_, state):
        lo, hi = state
        mid = lo + jax.lax.shift_right_logical(hi - lo, 1)
        big = _mxu_count(key >= mid, ones_bf) >= K
        return jnp.where(big, mid, lo), jnp.where(big, hi, mid)

    lo, hi = jax.lax.fori_loop(0, 16, bis_step, (lo0, hi0))

    r = K - _mxu_count(key >= hi, ones_bf)

    def ext_cond(state):
        r, _, _ = state
        return jnp.max(r) > 0

    def ext_step(state):
        r, cap, t = state
        inwin = jnp.logical_and(key >= lo, key < cap)
        m = jnp.max(jnp.where(inwin, key, INT_MIN), axis=1, keepdims=True)
        c = _mxu_count(key == m, ones_bf)
        live = r > 0
        t = jnp.where(live, m, t)
        cap = jnp.where(live, m, cap)
        return r - jnp.where(live, c, 0), cap, t

    r, _, t = jax.lax.while_loop(ext_cond, ext_step, (r, hi, lo))

    sparse = jnp.where(key >= t, lat, 0.0)
    sp_ref[...] = sparse

    rec = jax.lax.dot_general(
        sparse.astype(jnp.bfloat16), wd_ref[...], (((1,), (1,)), ((), ())),
        preferred_element_type=jnp.float32,
    ) + bd_ref[...]
    rec_ref[...] = rec


@jax.jit
def kernel(x, W_enc, b_enc, W_dec, b_dec):
    B = x.shape[0]
    grid = (B // TM,)
    out = pl.pallas_call(
        _body,
        grid=grid,
        in_specs=[
            pl.BlockSpec((TM, INPUT_DIM), lambda i: (i, 0)),
            pl.BlockSpec((LATENT_DIM, INPUT_DIM), lambda i: (0, 0)),
            pl.BlockSpec((1, LATENT_DIM), lambda i: (0, 0)),
            pl.BlockSpec((INPUT_DIM, LATENT_DIM), lambda i: (0, 0)),
            pl.BlockSpec((1, INPUT_DIM), lambda i: (0, 0)),
        ],
        out_specs=[
            pl.BlockSpec((TM, LATENT_DIM), lambda i: (i, 0)),
            pl.BlockSpec((TM, INPUT_DIM), lambda i: (i, 0)),
        ],
        out_shape=[
            jax.ShapeDtypeStruct((B, LATENT_DIM), jnp.float32),
            jax.ShapeDtypeStruct((B, INPUT_DIM), jnp.float32),
        ],
        compiler_params=pltpu.CompilerParams(
            vmem_limit_bytes=100 * 1024 * 1024,
        ),
    )(x, W_enc, b_enc.reshape(1, LATENT_DIM),
      W_dec.astype(jnp.bfloat16), b_dec.reshape(1, INPUT_DIM))
    sparse, recon = out
    return (recon, sparse)


# speedup vs baseline: 1.2671x; 1.2671x over previous
import jax
import jax.numpy as jnp
from jax.experimental import pallas as pl
from jax.experimental.pallas import tpu as pltpu

INPUT_DIM = 1024
LATENT_DIM = 4096
K = 64
TM = 256
NCHUNK = 64
WINDOW = 1 << 16

INT_MIN = -(2**31)


def _count(mask):
    return jnp.sum(mask.astype(jnp.int32), axis=1, keepdims=True)


def _body(x_ref, we_ref, be_ref, wd_ref, bd_ref, sp_ref, rec_ref):
    lat = jax.lax.dot_general(
        x_ref[...], we_ref[...], (((1,), (1,)), ((), ())),
        preferred_element_type=jnp.float32,
    ) + be_ref[...]

    ikey = jax.lax.bitcast_convert_type(lat, jnp.int32)
    key = jnp.where(ikey < 0, ikey ^ jnp.int32(0x7FFFFFFF), ikey)

    lo0 = jnp.full((TM, 1), INT_MIN, dtype=jnp.int32)
    hi0 = jnp.full((TM, 1), 2**31 - 1, dtype=jnp.int32)

    def bis_step(_, state):
        lo, hi = state
        mid = lo + jax.lax.shift_right_logical(hi - lo, 1)
        big = _count(key >= mid) >= K
        return jnp.where(big, mid, lo), jnp.where(big, hi, mid)

    lo, hi = jax.lax.fori_loop(0, 16, bis_step, (lo0, hi0))

    r = K - _count(key >= hi)

    def ext_cond(state):
        r, _, _ = state
        return jnp.max(r) > 0

    def ext_step(state):
        r, cap, t = state
        inwin = jnp.logical_and(key >= lo, key < cap)
        m = jnp.max(jnp.where(inwin, key, INT_MIN), axis=1, keepdims=True)
        c = _count(key == m)
        live = r > 0
        t = jnp.where(live, m, t)
        cap = jnp.where(live, m, cap)
        return r - jnp.where(live, c, 0), cap, t

    r, _, t = jax.lax.while_loop(ext_cond, ext_step, (r, hi, lo))

    sparse = jnp.where(key >= t, lat, 0.0)
    sp_ref[...] = sparse

    rec = jax.lax.dot_general(
        sparse.astype(jnp.bfloat16), wd_ref[...], (((1,), (1,)), ((), ())),
        preferred_element_type=jnp.float32,
    ) + bd_ref[...]
    rec_ref[...] = rec


@jax.jit
def kernel(x, W_enc, b_enc, W_dec, b_dec):
    B = x.shape[0]
    grid = (B // TM,)
    out = pl.pallas_call(
        _body,
        grid=grid,
        in_specs=[
            pl.BlockSpec((TM, INPUT_DIM), lambda i: (i, 0)),
            pl.BlockSpec((LATENT_DIM, INPUT_DIM), lambda i: (0, 0)),
            pl.BlockSpec((1, LATENT_DIM), lambda i: (0, 0)),
            pl.BlockSpec((INPUT_DIM, LATENT_DIM), lambda i: (0, 0)),
            pl.BlockSpec((1, INPUT_DIM), lambda i: (0, 0)),
        ],
        out_specs=[
            pl.BlockSpec((TM, LATENT_DIM), lambda i: (i, 0)),
            pl.BlockSpec((TM, INPUT_DIM), lambda i: (i, 0)),
        ],
        out_shape=[
            jax.ShapeDtypeStruct((B, LATENT_DIM), jnp.float32),
            jax.ShapeDtypeStruct((B, INPUT_DIM), jnp.float32),
        ],
        compiler_params=pltpu.CompilerParams(
            vmem_limit_bytes=100 * 1024 * 1024,
        ),
    )(x, W_enc, b_enc.reshape(1, LATENT_DIM),
      W_dec.astype(jnp.bfloat16), b_dec.reshape(1, INPUT_DIM))
    sparse, recon = out
    return (recon, sparse)
